# baseline (speedup 1.0000x reference)
import functools

import jax
import jax.numpy as jnp
from jax import lax
from jax.experimental import pallas as pl
from jax.experimental.pallas import tpu as pltpu
from jax.experimental.pallas import tpu_sc as plsc

N = 10000
B = 4
K = 32
ROW_TILE = 1000
GRID = N // ROW_TILE


def _elu(x):
    return jnp.where(x > 0, x, jnp.exp(x) - 1.0)


def _ln(x, g, b):
    mu = jnp.mean(x, axis=-1, keepdims=True)
    xc = x - mu
    var = jnp.mean(xc * xc, axis=-1, keepdims=True)
    return xc / jnp.sqrt(var + 1e-5) * g + b


def _dot(x, w):
    return jnp.dot(x, w, preferred_element_type=jnp.float32)


def _full_spec(shape):
    nd = len(shape)
    return pl.BlockSpec(shape, lambda i, _nd=nd: (0,) * _nd)


def _row_spec(d):
    return pl.BlockSpec((ROW_TILE, d), lambda i: (i, 0))


def _row_call(body, ins, consts, out_dims):
    in_specs = [_row_spec(a.shape[-1]) for a in ins]
    in_specs += [_full_spec(c.shape) for c in consts]
    if isinstance(out_dims, tuple):
        out_shape = tuple(jax.ShapeDtypeStruct((N, d), jnp.float32) for d in out_dims)
        out_specs = tuple(_row_spec(d) for d in out_dims)
    else:
        out_shape = jax.ShapeDtypeStruct((N, out_dims), jnp.float32)
        out_specs = _row_spec(out_dims)
    return pl.pallas_call(
        body,
        grid=(GRID,),
        in_specs=in_specs,
        out_specs=out_specs,
        out_shape=out_shape,
    )(*ins, *consts)


def _nn0_body(x_ref, w0, b0, w1, b1, w2, b2, w3, b3, o_ref):
    h = x_ref[...]
    h = _elu(_dot(h, w0[...]) + b0[...])
    h = _elu(_dot(h, w1[...]) + b1[...])
    h = _elu(_dot(h, w2[...]) + b2[...])
    o_ref[...] = _dot(h, w3[...]) + b3[...]


def _nn0(x, lins):
    consts = []
    for p in lins:
        consts += [p['w'], p['b'].reshape(1, -1)]
    return _row_call(_nn0_body, [x], consts, 128)


def _proj_body(e_ref, ws, bs, wh, bh, s_ref, h_ref, sq_ref):
    e = e_ref[...]
    s = _dot(e, ws[...]) + bs[...]
    s_ref[...] = s
    h = _dot(e, wh[...]) + bh[...]
    h_ref[...] = jnp.concatenate(
        [h, jnp.zeros((h.shape[0], 128 - h.shape[1]), jnp.float32)], axis=-1)
    sq_ref[...] = jnp.sum(s * s, axis=-1, keepdims=True)


def _proj(e, p):
    consts = [p['lin_s']['w'], p['lin_s']['b'].reshape(1, -1),
              p['lin_h']['w'], p['lin_h']['b'].reshape(1, -1)]
    return _row_call(_proj_body, [e], consts, (4, 128, 1))


def _gravout_body(e_ref, agg_ref, w, b, g, bb, o_ref):
    e = e_ref[...]
    xin = jnp.concatenate([e, agg_ref[...]], axis=-1)
    xn = _dot(xin, w[...]) + b[...]
    o_ref[...] = _ln(e + xn, g[...], bb[...])


def _gravout(e, agg, p):
    consts = [p['lin_out']['w'], p['lin_out']['b'].reshape(1, -1),
              p['ln']['g'].reshape(1, -1), p['ln']['b'].reshape(1, -1)]
    return _row_call(_gravout_body, [e, agg], consts, 128)


def _ffn_body(x_ref, *refs):
    o_ref = refs[-1]
    refs = refs[:-1]
    ws = refs[0:10]
    lns = refs[10:18]
    h = x_ref[...]
    for i in range(4):
        h = _elu(_dot(h, ws[2 * i][...]) + ws[2 * i + 1][...])
        h = _ln(h, lns[2 * i][...], lns[2 * i + 1][...])
    o_ref[...] = _dot(h, ws[8][...]) + ws[9][...]


def _ffn(x, p, dout):
    consts = []
    for lp in p['lins']:
        consts += [lp['w'], lp['b'].reshape(1, -1)]
    for lp in p['lns']:
        consts += [lp['g'].reshape(1, -1), lp['b'].reshape(1, -1)]
    return _row_call(_ffn_body, [x], consts, dout)


_BUF = 144
_RESEL_AT = 128
_GRP = 16
_NQ = _GRP * K // 128


def _merge16(ka, va, kb, vb):
    kbr = lax.rev(kb, (0,))
    vbr = lax.rev(vb, (0,))
    m = ka <= kbr
    kl = jnp.where(m, ka, kbr)
    vl = jnp.where(m, va, vbr)
    kh = jnp.where(m, kbr, ka)
    vh = jnp.where(m, vbr, va)
    kl, vl = plsc.sort_key_val(kl, vl)
    kh, vh = plsc.sort_key_val(kh, vh)
    return kl, vl, kh, vh


def _sc_knn_agg(s_t, sq, h, seg):
    mesh = plsc.VectorSubcoreMesh(core_axis_name="c", subcore_axis_name="s")
    INF = float(jnp.inf)

    @functools.partial(
        pl.kernel,
        out_type=jax.ShapeDtypeStruct((N * 64,), jnp.float32),
        mesh=mesh,
        compiler_params=pltpu.CompilerParams(needs_layout_passes=False),
        scratch_types=[
            pltpu.VMEM((N,), jnp.float32),
            pltpu.VMEM((N,), jnp.float32),
            pltpu.VMEM((N,), jnp.float32),
            pltpu.VMEM((N,), jnp.float32),
            pltpu.VMEM((N,), jnp.float32),
            pltpu.VMEM((16,), jnp.float32),
            pltpu.VMEM((_BUF,), jnp.float32),
            pltpu.VMEM((_BUF,), jnp.int32),
            pltpu.VMEM((_GRP * K,), jnp.float32),
            pltpu.VMEM((_NQ, 128), jnp.int32),
            pltpu.VMEM((_NQ, 128, 128), jnp.float32),
            pltpu.VMEM((_GRP * 64,), jnp.float32),
            pltpu.SemaphoreType.DMA,
        ],
    )
    def body(s_t_hbm, sq_hbm, h_hbm, seg_hbm, out_hbm,
             s0, s1, s2, s3, sqv, segv, d2b, idxb, wbuf, gidx, hbuf, aggb,
             sem):
        iota = lax.iota(jnp.int32, 16)
        wid = lax.axis_index("s") * 2 + lax.axis_index("c")
        pltpu.sync_copy(s_t_hbm.at[0], s0)
        pltpu.sync_copy(s_t_hbm.at[1], s1)
        pltpu.sync_copy(s_t_hbm.at[2], s2)
        pltpu.sync_copy(s_t_hbm.at[3], s3)
        pltpu.sync_copy(sq_hbm, sqv)
        pltpu.sync_copy(seg_hbm, segv)
        zz = jnp.zeros((16,), jnp.int32)
        for q in range(_NQ):
            for o in range(8):
                gidx[q, pl.ds(o * 16, 16)] = zz

        ev = wid // 8
        sl = wid % 8
        sv = segv[pl.ds(0, 16)]
        st = jnp.sum(jnp.where(iota == ev, sv, 0.0)).astype(jnp.int32)
        en = jnp.sum(jnp.where(iota == ev + 1, sv, 0.0)).astype(jnp.int32)
        cnt_ev = en - st
        chunk = (cnt_ev + 7) // 8
        my_st = st + sl * chunk
        my_en = jnp.minimum(my_st + chunk, en)
        n_my = jnp.maximum(my_en - my_st, 0)
        ngrp = (n_my + _GRP - 1) // _GRP
        st16 = (st // 16) * 16
        nvec = (en - st16 + 15) // 16

        def reselect(cnt):
            ak = jnp.full((16,), INF, jnp.float32)
            av = jnp.zeros((16,), jnp.int32)
            bk = jnp.full((16,), INF, jnp.float32)
            bv = jnp.zeros((16,), jnp.int32)
            for b in range(_BUF // 16):
                off = b * 16
                kb = jnp.where(iota + off < cnt, d2b[pl.ds(off, 16)], INF)
                vb = idxb[pl.ds(off, 16)]
                kb, vb = plsc.sort_key_val(kb, vb)
                if b == 0:
                    ak, av = kb, vb
                else:
                    ak, av, hk, hv = _merge16(ak, av, kb, vb)
                    bk, bv, _, _ = _merge16(bk, bv, hk, hv)
            d2b[pl.ds(0, 16)] = ak
            idxb[pl.ds(0, 16)] = av
            d2b[pl.ds(16, 16)] = bk
            idxb[pl.ds(16, 16)] = bv
            return jnp.int32(K), jnp.max(bk)

        def select_row(i, r):
            iv = jnp.full((16,), 0, jnp.int32) + i
            s0i = plsc.load_gather(s0, [iv])
            s1i = plsc.load_gather(s1, [iv])
            s2i = plsc.load_gather(s2, [iv])
            s3i = plsc.load_gather(s3, [iv])
            sqi = plsc.load_gather(sqv, [iv])

            def cbody(v, carry):
                cnt, tau = carry
                j = st16 + v * 16
                jv = j + iota
                a0 = s0[pl.ds(j, 16)]
                a1 = s1[pl.ds(j, 16)]
                a2 = s2[pl.ds(j, 16)]
                a3 = s3[pl.ds(j, 16)]
                sqj = sqv[pl.ds(j, 16)]
                t = a0 * s0i + a1 * s1i + a2 * s2i + a3 * s3i
                d2 = (sqi + sqj) - 2.0 * t
                valid = (jv >= st) & (jv < en)
                d2 = jnp.where(valid, d2, INF)
                m = d2 < tau
                pc = plsc.all_reduce_population_count(m)[0]

                def hit(cnt=cnt, tau=tau, d2=d2, jv=jv, m=m, pc=pc):
                    plsc.store_compressed(d2b.at[pl.ds(cnt, 16)], d2, mask=m)
                    plsc.store_compressed(idxb.at[pl.ds(cnt, 16)], jv, mask=m)
                    c2 = cnt + pc
                    return lax.cond(c2 > _RESEL_AT,
                                    lambda: reselect(c2),
                                    lambda: (c2, tau))

                return lax.cond(pc > 0, hit,
                                lambda c=cnt, t=tau: (c, t))

            cnt, _ = lax.fori_loop(0, nvec, cbody, (jnp.int32(0), INF))
            reselect(cnt)
            q = r // 4
            for half in range(2):
                vv = idxb[pl.ds(half * 16, 16)]
                g0 = plsc.load_gather(s0, [vv])
                g1 = plsc.load_gather(s1, [vv])
                g2 = plsc.load_gather(s2, [vv])
                g3 = plsc.load_gather(s3, [vv])
                dd = ((s0i - g0) * (s0i - g0) + (s1i - g1) * (s1i - g1)
                      + (s2i - g2) * (s2i - g2) + (s3i - g3) * (s3i - g3))
                w = jnp.exp(-10.0 * dd)
                wbuf[pl.ds(r * K + half * 16, 16)] = w
                gidx[q, pl.ds((r % 4) * K + half * 16, 16)] = vv

        def agg_row(r):
            zf = jnp.zeros((16,), jnp.float32)
            ninf = jnp.full((16,), -INF, jnp.float32)

            def kbody(k, carry):
                acc0, acc1, mx0, mx1 = carry
                slot = r * K + k
                q = slot // 128
                rr = slot % 128
                h0 = hbuf[q, rr, pl.ds(0, 16)]
                h1 = hbuf[q, rr, pl.ds(16, 16)]
                wk = plsc.load_gather(wbuf, [jnp.full((16,), 0, jnp.int32) + slot])
                m0 = h0 * wk
                m1 = h1 * wk
                return (acc0 + m0, acc1 + m1,
                        jnp.maximum(mx0, m0), jnp.maximum(mx1, m1))

            acc0, acc1, mx0, mx1 = lax.fori_loop(
                0, K, kbody, (zf, zf, ninf, ninf))
            sc = jnp.float32(1.0 / K)
            aggb[pl.ds(r * 64, 16)] = acc0 * sc
            aggb[pl.ds(r * 64 + 16, 16)] = acc1 * sc
            aggb[pl.ds(r * 64 + 32, 16)] = mx0
            aggb[pl.ds(r * 64 + 48, 16)] = mx1

        def gbody(g, _):
            base = my_st + g * _GRP
            nr = jnp.minimum(my_en - base, _GRP)

            def rbody(r, _):
                select_row(base + r, r)
                return 0

            lax.fori_loop(0, nr, rbody, 0)
            copies = [pltpu.async_copy(h_hbm.at[gidx.at[q]], hbuf.at[q], sem)
                      for q in range(_NQ)]
            for c in copies:
                c.wait()

            def abody(r, _):
                agg_row(r)
                return 0

            lax.fori_loop(0, nr, abody, 0)

            @pl.when(nr == _GRP)
            def _():
                pltpu.sync_copy(aggb, out_hbm.at[pl.ds(base * 64, _GRP * 64)])

            @pl.when(nr < _GRP)
            def _():
                def cb(r, _):
                    pltpu.sync_copy(aggb.at[pl.ds(r * 64, 64)],
                                    out_hbm.at[pl.ds((base + r) * 64, 64)])
                    return 0
                lax.fori_loop(0, nr, cb, 0)

            return 0

        lax.fori_loop(0, ngrp, gbody, 0)

    return body(s_t, sq, h, seg)


def _gravnet_layer(p, e, batch_index, seg):
    s, hh, sq = _proj(e, p)
    agg = _sc_knn_agg(s.T, sq[:, 0], hh, seg).reshape(N, 64)
    return _gravout(e, agg, p)


def kernel(x, params, batch_index):
    starts = jnp.searchsorted(batch_index, jnp.arange(B, dtype=jnp.int32),
                              side='left').astype(jnp.float32)
    seg = jnp.concatenate(
        [starts, jnp.full((12,), N, dtype=jnp.float32)])
    emb = _nn0(x, params['nn0'])
    e = emb
    embs_id = []
    for p in params['conv_id']:
        e = _gravnet_layer(p, e, batch_index, seg)
        embs_id.append(e)
    e = emb
    embs_reg = []
    for p in params['conv_reg']:
        e = _gravnet_layer(p, e, batch_index, seg)
        embs_reg.append(e)
    embedding_id = jnp.concatenate([x] + embs_id, axis=-1)
    preds_id = _ffn(embedding_id, params['nn_id'], 6)
    embedding_reg = jnp.concatenate([x] + embs_reg + [preds_id], axis=-1)
    preds_pt = _ffn(embedding_reg, params['nn_pt'], 1) + x[:, 1:2]
    preds_eta = _ffn(embedding_reg, params['nn_eta'], 1) + x[:, 2:3]
    preds_phi = _ffn(embedding_reg, params['nn_phi'], 1) + x[:, 3:4]
    preds_energy = _ffn(embedding_reg, params['nn_energy'], 1) + x[:, 4:5]
    preds_momentum = jnp.concatenate(
        [preds_pt, preds_eta, preds_phi, preds_energy], axis=-1)
    pred_charge = _ffn(embedding_reg, params['nn_charge'], 3)
    return (preds_id, preds_momentum, pred_charge)


# speedup vs baseline: 1.1805x; 1.1805x over previous
import functools

import jax
import jax.numpy as jnp
from jax import lax
from jax.experimental import pallas as pl
from jax.experimental.pallas import tpu as pltpu
from jax.experimental.pallas import tpu_sc as plsc

N = 10000
B = 4
K = 32
ROW_TILE = 1000
GRID = N // ROW_TILE


def _elu(x):
    return jnp.where(x > 0, x, jnp.exp(x) - 1.0)


def _ln(x, g, b):
    mu = jnp.mean(x, axis=-1, keepdims=True)
    xc = x - mu
    var = jnp.mean(xc * xc, axis=-1, keepdims=True)
    return xc / jnp.sqrt(var + 1e-5) * g + b


def _dot(x, w):
    return jnp.dot(x, w, preferred_element_type=jnp.float32)


def _full_spec(shape):
    nd = len(shape)
    return pl.BlockSpec(shape, lambda i, _nd=nd: (0,) * _nd)


def _row_spec(d):
    return pl.BlockSpec((ROW_TILE, d), lambda i: (i, 0))


def _row_call(body, ins, consts, out_dims):
    in_specs = [_row_spec(a.shape[-1]) for a in ins]
    in_specs += [_full_spec(c.shape) for c in consts]
    if isinstance(out_dims, tuple):
        out_shape = tuple(jax.ShapeDtypeStruct((N, d), jnp.float32) for d in out_dims)
        out_specs = tuple(_row_spec(d) for d in out_dims)
    else:
        out_shape = jax.ShapeDtypeStruct((N, out_dims), jnp.float32)
        out_specs = _row_spec(out_dims)
    return pl.pallas_call(
        body,
        grid=(GRID,),
        in_specs=in_specs,
        out_specs=out_specs,
        out_shape=out_shape,
    )(*ins, *consts)


def _nn0_body(x_ref, w0, b0, w1, b1, w2, b2, w3, b3, o_ref):
    h = x_ref[...]
    h = _elu(_dot(h, w0[...]) + b0[...])
    h = _elu(_dot(h, w1[...]) + b1[...])
    h = _elu(_dot(h, w2[...]) + b2[...])
    o_ref[...] = _dot(h, w3[...]) + b3[...]


def _nn0(x, lins):
    consts = []
    for p in lins:
        consts += [p['w'], p['b'].reshape(1, -1)]
    return _row_call(_nn0_body, [x], consts, 128)


def _proj_body(e_ref, ws, bs, wh, bh, s_ref, h_ref, sq_ref):
    e = e_ref[...]
    s = _dot(e, ws[...]) + bs[...]
    s_ref[...] = s
    h = _dot(e, wh[...]) + bh[...]
    h_ref[...] = jnp.concatenate(
        [h, jnp.zeros((h.shape[0], 128 - h.shape[1]), jnp.float32)], axis=-1)
    sq_ref[...] = jnp.sum(s * s, axis=-1, keepdims=True)


def _proj(e, p):
    consts = [p['lin_s']['w'], p['lin_s']['b'].reshape(1, -1),
              p['lin_h']['w'], p['lin_h']['b'].reshape(1, -1)]
    return _row_call(_proj_body, [e], consts, (4, 128, 1))


def _gravout_body(e_ref, agg_ref, w, b, g, bb, o_ref):
    e = e_ref[...]
    xin = jnp.concatenate([e, agg_ref[...]], axis=-1)
    xn = _dot(xin, w[...]) + b[...]
    o_ref[...] = _ln(e + xn, g[...], bb[...])


def _gravout(e, agg, p):
    consts = [p['lin_out']['w'], p['lin_out']['b'].reshape(1, -1),
              p['ln']['g'].reshape(1, -1), p['ln']['b'].reshape(1, -1)]
    return _row_call(_gravout_body, [e, agg], consts, 128)


def _ffn_body(x_ref, *refs):
    o_ref = refs[-1]
    refs = refs[:-1]
    ws = refs[0:10]
    lns = refs[10:18]
    h = x_ref[...]
    for i in range(4):
        h = _elu(_dot(h, ws[2 * i][...]) + ws[2 * i + 1][...])
        h = _ln(h, lns[2 * i][...], lns[2 * i + 1][...])
    o_ref[...] = _dot(h, ws[8][...]) + ws[9][...]


def _ffn(x, p, dout):
    consts = []
    for lp in p['lins']:
        consts += [lp['w'], lp['b'].reshape(1, -1)]
    for lp in p['lns']:
        consts += [lp['g'].reshape(1, -1), lp['b'].reshape(1, -1)]
    return _row_call(_ffn_body, [x], consts, dout)


_BUF = 144
_RESEL_AT = 128
_GRP = 16
_NQ = _GRP * K // 128


def _merge16(ka, va, kb, vb):
    kbr = lax.rev(kb, (0,))
    vbr = lax.rev(vb, (0,))
    m = ka <= kbr
    kl = jnp.where(m, ka, kbr)
    vl = jnp.where(m, va, vbr)
    kh = jnp.where(m, kbr, ka)
    vh = jnp.where(m, vbr, va)
    kl, vl = plsc.sort_key_val(kl, vl)
    kh, vh = plsc.sort_key_val(kh, vh)
    return kl, vl, kh, vh


def _sc_knn_agg(s_t, sq, h, seg):
    mesh = plsc.VectorSubcoreMesh(core_axis_name="c", subcore_axis_name="s")
    INF = float(jnp.inf)

    @functools.partial(
        pl.kernel,
        out_type=jax.ShapeDtypeStruct((N * 64,), jnp.float32),
        mesh=mesh,
        compiler_params=pltpu.CompilerParams(needs_layout_passes=False),
        scratch_types=[
            pltpu.VMEM((N,), jnp.float32),
            pltpu.VMEM((N,), jnp.float32),
            pltpu.VMEM((N,), jnp.float32),
            pltpu.VMEM((N,), jnp.float32),
            pltpu.VMEM((N,), jnp.float32),
            pltpu.VMEM((16,), jnp.float32),
            pltpu.VMEM((_BUF,), jnp.float32),
            pltpu.VMEM((_BUF,), jnp.int32),
            pltpu.VMEM((_GRP * K,), jnp.float32),
            pltpu.VMEM((_NQ, 128), jnp.int32),
            pltpu.VMEM((_NQ, 128, 128), jnp.float32),
            pltpu.VMEM((_GRP * 64,), jnp.float32),
            pltpu.SemaphoreType.DMA,
        ],
    )
    def body(s_t_hbm, sq_hbm, h_hbm, seg_hbm, out_hbm,
             s0, s1, s2, s3, sqv, segv, d2b, idxb, wbuf, gidx, hbuf, aggb,
             sem):
        iota = lax.iota(jnp.int32, 16)
        wid = lax.axis_index("s") * 2 + lax.axis_index("c")
        pltpu.sync_copy(s_t_hbm.at[0], s0)
        pltpu.sync_copy(s_t_hbm.at[1], s1)
        pltpu.sync_copy(s_t_hbm.at[2], s2)
        pltpu.sync_copy(s_t_hbm.at[3], s3)
        pltpu.sync_copy(sq_hbm, sqv)
        pltpu.sync_copy(seg_hbm, segv)
        zz = jnp.zeros((16,), jnp.int32)
        for q in range(_NQ):
            for o in range(8):
                gidx[q, pl.ds(o * 16, 16)] = zz

        ev = wid // 8
        sl = wid % 8
        sv = segv[pl.ds(0, 16)]
        st = jnp.sum(jnp.where(iota == ev, sv, 0.0)).astype(jnp.int32)
        en = jnp.sum(jnp.where(iota == ev + 1, sv, 0.0)).astype(jnp.int32)
        cnt_ev = en - st
        chunk = (cnt_ev + 7) // 8
        my_st = st + sl * chunk
        my_en = jnp.minimum(my_st + chunk, en)
        n_my = jnp.maximum(my_en - my_st, 0)
        ngrp = (n_my + _GRP - 1) // _GRP
        st16 = (st // 16) * 16
        nvec = (en - st16 + 15) // 16

        def reselect(cnt):
            ak = jnp.full((16,), INF, jnp.float32)
            av = jnp.zeros((16,), jnp.int32)
            bk = jnp.full((16,), INF, jnp.float32)
            bv = jnp.zeros((16,), jnp.int32)
            for b in range(_BUF // 16):
                off = b * 16
                kb = jnp.where(iota + off < cnt, d2b[pl.ds(off, 16)], INF)
                vb = idxb[pl.ds(off, 16)]
                kb, vb = plsc.sort_key_val(kb, vb)
                if b == 0:
                    ak, av = kb, vb
                else:
                    ak, av, hk, hv = _merge16(ak, av, kb, vb)
                    bk, bv, _, _ = _merge16(bk, bv, hk, hv)
            d2b[pl.ds(0, 16)] = ak
            idxb[pl.ds(0, 16)] = av
            d2b[pl.ds(16, 16)] = bk
            idxb[pl.ds(16, 16)] = bv
            return jnp.int32(K), jnp.max(bk)

        def select_row(i, r):
            iv = jnp.full((16,), 0, jnp.int32) + i
            s0i = plsc.load_gather(s0, [iv])
            s1i = plsc.load_gather(s1, [iv])
            s2i = plsc.load_gather(s2, [iv])
            s3i = plsc.load_gather(s3, [iv])
            sqi = plsc.load_gather(sqv, [iv])

            def cbody(v, carry):
                cnt, tau = carry
                j = st16 + v * 16
                jv = j + iota
                a0 = s0[pl.ds(j, 16)]
                a1 = s1[pl.ds(j, 16)]
                a2 = s2[pl.ds(j, 16)]
                a3 = s3[pl.ds(j, 16)]
                sqj = sqv[pl.ds(j, 16)]
                t = a0 * s0i + a1 * s1i + a2 * s2i + a3 * s3i
                d2 = (sqi + sqj) - 2.0 * t
                valid = (jv >= st) & (jv < en)
                d2 = jnp.where(valid, d2, INF)
                m = d2 < tau
                plsc.store_compressed(d2b.at[pl.ds(cnt, 16)], d2, mask=m)
                plsc.store_compressed(idxb.at[pl.ds(cnt, 16)], jv, mask=m)
                cnt = cnt + plsc.all_reduce_population_count(m)[0]
                cnt, tau = lax.cond(cnt > _RESEL_AT,
                                    lambda c=cnt: reselect(c),
                                    lambda c=cnt, t=tau: (c, t))
                return cnt, tau

            cnt, _ = lax.fori_loop(0, nvec, cbody, (jnp.int32(0), INF))
            reselect(cnt)
            q = r // 4
            for half in range(2):
                vv = idxb[pl.ds(half * 16, 16)]
                g0 = plsc.load_gather(s0, [vv])
                g1 = plsc.load_gather(s1, [vv])
                g2 = plsc.load_gather(s2, [vv])
                g3 = plsc.load_gather(s3, [vv])
                dd = ((s0i - g0) * (s0i - g0) + (s1i - g1) * (s1i - g1)
                      + (s2i - g2) * (s2i - g2) + (s3i - g3) * (s3i - g3))
                w = jnp.exp(-10.0 * dd)
                wbuf[pl.ds(r * K + half * 16, 16)] = w
                gidx[q, pl.ds((r % 4) * K + half * 16, 16)] = vv

        def agg_row(r):
            zf = jnp.zeros((16,), jnp.float32)
            ninf = jnp.full((16,), -INF, jnp.float32)

            def kbody(k, carry):
                acc0, acc1, mx0, mx1 = carry
                slot = r * K + k
                q = slot // 128
                rr = slot % 128
                h0 = hbuf[q, rr, pl.ds(0, 16)]
                h1 = hbuf[q, rr, pl.ds(16, 16)]
                wk = plsc.load_gather(wbuf, [jnp.full((16,), 0, jnp.int32) + slot])
                m0 = h0 * wk
                m1 = h1 * wk
                return (acc0 + m0, acc1 + m1,
                        jnp.maximum(mx0, m0), jnp.maximum(mx1, m1))

            acc0, acc1, mx0, mx1 = lax.fori_loop(
                0, K, kbody, (zf, zf, ninf, ninf))
            sc = jnp.float32(1.0 / K)
            aggb[pl.ds(r * 64, 16)] = acc0 * sc
            aggb[pl.ds(r * 64 + 16, 16)] = acc1 * sc
            aggb[pl.ds(r * 64 + 32, 16)] = mx0
            aggb[pl.ds(r * 64 + 48, 16)] = mx1

        def gbody(g, _):
            base = my_st + g * _GRP
            nr = jnp.minimum(my_en - base, _GRP)

            copies = []
            for q in range(_NQ):
                def rbody(r, _, q=q):
                    rr = q * 4 + r

                    @pl.when(rr < nr)
                    def _():
                        select_row(base + rr, rr)
                    return 0

                lax.fori_loop(0, 4, rbody, 0)
                copies.append(
                    pltpu.async_copy(h_hbm.at[gidx.at[q]], hbuf.at[q], sem))
            for c in copies:
                c.wait()

            def abody(r, _):
                agg_row(r)
                return 0

            lax.fori_loop(0, nr, abody, 0)

            @pl.when(nr == _GRP)
            def _():
                pltpu.sync_copy(aggb, out_hbm.at[pl.ds(base * 64, _GRP * 64)])

            @pl.when(nr < _GRP)
            def _():
                def cb(r, _):
                    pltpu.sync_copy(aggb.at[pl.ds(r * 64, 64)],
                                    out_hbm.at[pl.ds((base + r) * 64, 64)])
                    return 0
                lax.fori_loop(0, nr, cb, 0)

            return 0

        lax.fori_loop(0, ngrp, gbody, 0)

    return body(s_t, sq, h, seg)


def _gravnet_layer(p, e, batch_index, seg):
    s, hh, sq = _proj(e, p)
    agg = _sc_knn_agg(s.T, sq[:, 0], hh, seg).reshape(N, 64)
    return _gravout(e, agg, p)


def kernel(x, params, batch_index):
    starts = jnp.searchsorted(batch_index, jnp.arange(B, dtype=jnp.int32),
                              side='left').astype(jnp.float32)
    seg = jnp.concatenate(
        [starts, jnp.full((12,), N, dtype=jnp.float32)])
    emb = _nn0(x, params['nn0'])
    e = emb
    embs_id = []
    for p in params['conv_id']:
        e = _gravnet_layer(p, e, batch_index, seg)
        embs_id.append(e)
    e = emb
    embs_reg = []
    for p in params['conv_reg']:
        e = _gravnet_layer(p, e, batch_index, seg)
        embs_reg.append(e)
    embedding_id = jnp.concatenate([x] + embs_id, axis=-1)
    preds_id = _ffn(embedding_id, params['nn_id'], 6)
    embedding_reg = jnp.concatenate([x] + embs_reg + [preds_id], axis=-1)
    preds_pt = _ffn(embedding_reg, params['nn_pt'], 1) + x[:, 1:2]
    preds_eta = _ffn(embedding_reg, params['nn_eta'], 1) + x[:, 2:3]
    preds_phi = _ffn(embedding_reg, params['nn_phi'], 1) + x[:, 3:4]
    preds_energy = _ffn(embedding_reg, params['nn_energy'], 1) + x[:, 4:5]
    preds_momentum = jnp.concatenate(
        [preds_pt, preds_eta, preds_phi, preds_energy], axis=-1)
    pred_charge = _ffn(embedding_reg, params['nn_charge'], 3)
    return (preds_id, preds_momentum, pred_charge)


# speedup vs baseline: 2.3794x; 2.0156x over previous
import functools

import jax
import jax.numpy as jnp
from jax import lax
from jax.experimental import pallas as pl
from jax.experimental.pallas import tpu as pltpu
from jax.experimental.pallas import tpu_sc as plsc

N = 10000
B = 4
K = 32
SPACE_DIMS = 4
ROW_TILE = 1000
GRID = N // ROW_TILE


def _elu(x):
    return jnp.where(x > 0, x, jnp.exp(x) - 1.0)


def _ln(x, g, b):
    mu = jnp.mean(x, axis=-1, keepdims=True)
    xc = x - mu
    var = jnp.mean(xc * xc, axis=-1, keepdims=True)
    return xc / jnp.sqrt(var + 1e-5) * g + b


def _dot(x, w):
    return jnp.dot(x, w, preferred_element_type=jnp.float32)


def _full_spec(shape):
    nd = len(shape)
    return pl.BlockSpec(shape, lambda i, _nd=nd: (0,) * _nd)


def _row_spec(d):
    return pl.BlockSpec((ROW_TILE, d), lambda i: (i, 0))


def _row_call(body, ins, consts, out_dims):
    in_specs = [_row_spec(a.shape[-1]) for a in ins]
    in_specs += [_full_spec(c.shape) for c in consts]
    if isinstance(out_dims, tuple):
        out_shape = tuple(jax.ShapeDtypeStruct((N, d), jnp.float32) for d in out_dims)
        out_specs = tuple(_row_spec(d) for d in out_dims)
    else:
        out_shape = jax.ShapeDtypeStruct((N, out_dims), jnp.float32)
        out_specs = _row_spec(out_dims)
    return pl.pallas_call(
        body,
        grid=(GRID,),
        in_specs=in_specs,
        out_specs=out_specs,
        out_shape=out_shape,
    )(*ins, *consts)


def _nn0_body(x_ref, w0, b0, w1, b1, w2, b2, w3, b3, o_ref):
    h = x_ref[...]
    h = _elu(_dot(h, w0[...]) + b0[...])
    h = _elu(_dot(h, w1[...]) + b1[...])
    h = _elu(_dot(h, w2[...]) + b2[...])
    o_ref[...] = _dot(h, w3[...]) + b3[...]


def _nn0(x, lins):
    consts = []
    for p in lins:
        consts += [p['w'], p['b'].reshape(1, -1)]
    return _row_call(_nn0_body, [x], consts, 128)


def _proj_body(e_ref, ws, bs, wh, bh, s_ref, h_ref, sq_ref):
    e = e_ref[...]
    s = _dot(e, ws[...]) + bs[...]
    s_ref[...] = s
    h = _dot(e, wh[...]) + bh[...]
    h_ref[...] = jnp.concatenate(
        [h, jnp.zeros((h.shape[0], 128 - h.shape[1]), jnp.float32)], axis=-1)
    sq_ref[...] = jnp.sum(s * s, axis=-1, keepdims=True)


def _proj(e, p):
    consts = [p['lin_s']['w'], p['lin_s']['b'].reshape(1, -1),
              p['lin_h']['w'], p['lin_h']['b'].reshape(1, -1)]
    return _row_call(_proj_body, [e], consts, (4, 128, 1))


def _gravout_body(e_ref, agg_ref, w, b, g, bb, o_ref):
    e = e_ref[...]
    xin = jnp.concatenate([e, agg_ref[...]], axis=-1)
    xn = _dot(xin, w[...]) + b[...]
    o_ref[...] = _ln(e + xn, g[...], bb[...])


def _gravout(e, agg, p):
    consts = [p['lin_out']['w'], p['lin_out']['b'].reshape(1, -1),
              p['ln']['g'].reshape(1, -1), p['ln']['b'].reshape(1, -1)]
    return _row_call(_gravout_body, [e, agg], consts, 128)


def _ffn_body(x_ref, *refs):
    o_ref = refs[-1]
    refs = refs[:-1]
    ws = refs[0:10]
    lns = refs[10:18]
    h = x_ref[...]
    for i in range(4):
        h = _elu(_dot(h, ws[2 * i][...]) + ws[2 * i + 1][...])
        h = _ln(h, lns[2 * i][...], lns[2 * i + 1][...])
    o_ref[...] = _dot(h, ws[8][...]) + ws[9][...]


def _ffn(x, p, dout):
    consts = []
    for lp in p['lins']:
        consts += [lp['w'], lp['b'].reshape(1, -1)]
    for lp in p['lns']:
        consts += [lp['g'].reshape(1, -1), lp['b'].reshape(1, -1)]
    return _row_call(_ffn_body, [x], consts, dout)


_BUF = 192
_RESEL_AT = 128
_NPAD = N + 64
_GRP = 16
_NQ = _GRP * K // 128


def _merge16(ka, va, kb, vb):
    kbr = lax.rev(kb, (0,))
    vbr = lax.rev(vb, (0,))
    m = ka <= kbr
    kl = jnp.where(m, ka, kbr)
    vl = jnp.where(m, va, vbr)
    kh = jnp.where(m, kbr, ka)
    vh = jnp.where(m, vbr, va)
    kl, vl = plsc.sort_key_val(kl, vl)
    kh, vh = plsc.sort_key_val(kh, vh)
    return kl, vl, kh, vh


def _sc_knn_agg(s_t, sq, h, seg):
    mesh = plsc.VectorSubcoreMesh(core_axis_name="c", subcore_axis_name="s")
    INF = float(jnp.inf)

    @functools.partial(
        pl.kernel,
        out_type=jax.ShapeDtypeStruct((N * 64,), jnp.float32),
        mesh=mesh,
        compiler_params=pltpu.CompilerParams(needs_layout_passes=False),
        scratch_types=[
            pltpu.VMEM((_NPAD,), jnp.float32),
            pltpu.VMEM((_NPAD,), jnp.float32),
            pltpu.VMEM((_NPAD,), jnp.float32),
            pltpu.VMEM((_NPAD,), jnp.float32),
            pltpu.VMEM((_NPAD,), jnp.float32),
            pltpu.VMEM((16,), jnp.float32),
            pltpu.VMEM((_BUF,), jnp.float32),
            pltpu.VMEM((_BUF,), jnp.int32),
            pltpu.VMEM((_GRP * K,), jnp.float32),
            pltpu.VMEM((_NQ, 128), jnp.int32),
            pltpu.VMEM((_NQ, 128, 128), jnp.float32),
            pltpu.VMEM((_GRP * 64,), jnp.float32),
            pltpu.SemaphoreType.DMA,
        ],
    )
    def body(s_t_hbm, sq_hbm, h_hbm, seg_hbm, out_hbm,
             s0, s1, s2, s3, sqv, segv, d2b, idxb, wbuf, gidx, hbuf, aggb,
             sem):
        iota = lax.iota(jnp.int32, 16)
        wid = lax.axis_index("s") * 2 + lax.axis_index("c")
        pltpu.sync_copy(s_t_hbm.at[0], s0)
        pltpu.sync_copy(s_t_hbm.at[1], s1)
        pltpu.sync_copy(s_t_hbm.at[2], s2)
        pltpu.sync_copy(s_t_hbm.at[3], s3)
        pltpu.sync_copy(sq_hbm, sqv)
        pltpu.sync_copy(seg_hbm, segv)
        zz = jnp.zeros((16,), jnp.int32)
        for q in range(_NQ):
            for o in range(8):
                gidx[q, pl.ds(o * 16, 16)] = zz

        ev = wid // 8
        sl = wid % 8
        sv = segv[pl.ds(0, 16)]
        st = jnp.sum(jnp.where(iota == ev, sv, 0.0)).astype(jnp.int32)
        en = jnp.sum(jnp.where(iota == ev + 1, sv, 0.0)).astype(jnp.int32)
        cnt_ev = en - st
        chunk = (cnt_ev + 7) // 8
        my_st = st + sl * chunk
        my_en = jnp.minimum(my_st + chunk, en)
        n_my = jnp.maximum(my_en - my_st, 0)
        ngrp = (n_my + _GRP - 1) // _GRP
        st16 = (st // 16) * 16
        nvec4 = (en - st16 + 63) // 64

        def reselect(cnt):
            ak = jnp.full((16,), INF, jnp.float32)
            av = jnp.zeros((16,), jnp.int32)
            bk = jnp.full((16,), INF, jnp.float32)
            bv = jnp.zeros((16,), jnp.int32)
            for b in range(_BUF // 16):
                off = b * 16
                kb = jnp.where(iota + off < cnt, d2b[pl.ds(off, 16)], INF)
                vb = idxb[pl.ds(off, 16)]
                kb, vb = plsc.sort_key_val(kb, vb)
                if b == 0:
                    ak, av = kb, vb
                else:
                    ak, av, hk, hv = _merge16(ak, av, kb, vb)
                    bk, bv, _, _ = _merge16(bk, bv, hk, hv)
            d2b[pl.ds(0, 16)] = ak
            idxb[pl.ds(0, 16)] = av
            d2b[pl.ds(16, 16)] = bk
            idxb[pl.ds(16, 16)] = bv
            return jnp.int32(K), jnp.max(bk)

        def select_row(i, r):
            iv = jnp.full((16,), 0, jnp.int32) + i
            s0i = plsc.load_gather(s0, [iv])
            s1i = plsc.load_gather(s1, [iv])
            s2i = plsc.load_gather(s2, [iv])
            s3i = plsc.load_gather(s3, [iv])
            sqi = plsc.load_gather(sqv, [iv])

            def cbody(v, carry):
                cnt, tau = carry
                jbase = st16 + v * 64
                d2s, jvs, ms = [], [], []
                for u in range(4):
                    j = jbase + u * 16
                    jv = j + iota
                    a0 = s0[pl.ds(j, 16)]
                    a1 = s1[pl.ds(j, 16)]
                    a2 = s2[pl.ds(j, 16)]
                    a3 = s3[pl.ds(j, 16)]
                    sqj = sqv[pl.ds(j, 16)]
                    t = a0 * s0i + a1 * s1i + a2 * s2i + a3 * s3i
                    d2 = (sqi + sqj) - 2.0 * t
                    valid = (jv >= st) & (jv < en)
                    d2 = jnp.where(valid, d2, INF)
                    d2s.append(d2)
                    jvs.append(jv)
                    ms.append(d2 < tau)
                pcs = [plsc.all_reduce_population_count(m)[0] for m in ms]
                offs = [cnt, cnt + pcs[0], cnt + pcs[0] + pcs[1],
                        cnt + pcs[0] + pcs[1] + pcs[2]]
                for u in range(4):
                    plsc.store_compressed(d2b.at[pl.ds(offs[u], 16)],
                                          d2s[u], mask=ms[u])
                    plsc.store_compressed(idxb.at[pl.ds(offs[u], 16)],
                                          jvs[u], mask=ms[u])
                cnt = offs[3] + pcs[3]
                cnt, tau = lax.cond(cnt > _RESEL_AT,
                                    lambda c=cnt: reselect(c),
                                    lambda c=cnt, t=tau: (c, t))
                return cnt, tau

            cnt, _ = lax.fori_loop(0, nvec4, cbody, (jnp.int32(0), INF))
            reselect(cnt)
            q = r // 4
            for half in range(2):
                vv = idxb[pl.ds(half * 16, 16)]
                g0 = plsc.load_gather(s0, [vv])
                g1 = plsc.load_gather(s1, [vv])
                g2 = plsc.load_gather(s2, [vv])
                g3 = plsc.load_gather(s3, [vv])
                dd = ((s0i - g0) * (s0i - g0) + (s1i - g1) * (s1i - g1)
                      + (s2i - g2) * (s2i - g2) + (s3i - g3) * (s3i - g3))
                w = jnp.exp(-10.0 * dd)
                wbuf[pl.ds(r * K + half * 16, 16)] = w
                gidx[q, pl.ds((r % 4) * K + half * 16, 16)] = vv

        def agg_row(r):
            zf = jnp.zeros((16,), jnp.float32)
            ninf = jnp.full((16,), -INF, jnp.float32)

            def kbody(k, carry):
                acc0, acc1, mx0, mx1 = carry
                slot = r * K + k
                q = slot // 128
                rr = slot % 128
                h0 = hbuf[q, rr, pl.ds(0, 16)]
                h1 = hbuf[q, rr, pl.ds(16, 16)]
                wk = plsc.load_gather(wbuf, [jnp.full((16,), 0, jnp.int32) + slot])
                m0 = h0 * wk
                m1 = h1 * wk
                return (acc0 + m0, acc1 + m1,
                        jnp.maximum(mx0, m0), jnp.maximum(mx1, m1))

            acc0, acc1, mx0, mx1 = lax.fori_loop(
                0, K, kbody, (zf, zf, ninf, ninf))
            sc = jnp.float32(1.0 / K)
            aggb[pl.ds(r * 64, 16)] = acc0 * sc
            aggb[pl.ds(r * 64 + 16, 16)] = acc1 * sc
            aggb[pl.ds(r * 64 + 32, 16)] = mx0
            aggb[pl.ds(r * 64 + 48, 16)] = mx1

        def gbody(g, _):
            base = my_st + g * _GRP
            nr = jnp.minimum(my_en - base, _GRP)

            copies = []
            for q in range(_NQ):
                def rbody(r, _, q=q):
                    rr = q * 4 + r

                    @pl.when(rr < nr)
                    def _():
                        select_row(base + rr, rr)
                    return 0

                lax.fori_loop(0, 4, rbody, 0)
                copies.append(
                    pltpu.async_copy(h_hbm.at[gidx.at[q]], hbuf.at[q], sem))
            for c in copies:
                c.wait()

            def abody(r, _):
                agg_row(r)
                return 0

            lax.fori_loop(0, nr, abody, 0)

            @pl.when(nr == _GRP)
            def _():
                pltpu.sync_copy(aggb, out_hbm.at[pl.ds(base * 64, _GRP * 64)])

            @pl.when(nr < _GRP)
            def _():
                def cb(r, _):
                    pltpu.sync_copy(aggb.at[pl.ds(r * 64, 64)],
                                    out_hbm.at[pl.ds((base + r) * 64, 64)])
                    return 0
                lax.fori_loop(0, nr, cb, 0)

            return 0

        lax.fori_loop(0, ngrp, gbody, 0)

    return body(s_t, sq, h, seg)


def _gravnet_layer(p, e, batch_index, seg):
    s, hh, sq = _proj(e, p)
    s_t = jnp.concatenate(
        [s.T, jnp.zeros((SPACE_DIMS, _NPAD - N), jnp.float32)], axis=1)
    sq_p = jnp.concatenate(
        [sq[:, 0], jnp.zeros((_NPAD - N,), jnp.float32)])
    agg = _sc_knn_agg(s_t, sq_p, hh, seg).reshape(N, 64)
    return _gravout(e, agg, p)


def kernel(x, params, batch_index):
    starts = jnp.searchsorted(batch_index, jnp.arange(B, dtype=jnp.int32),
                              side='left').astype(jnp.float32)
    seg = jnp.concatenate(
        [starts, jnp.full((12,), N, dtype=jnp.float32)])
    emb = _nn0(x, params['nn0'])
    e = emb
    embs_id = []
    for p in params['conv_id']:
        e = _gravnet_layer(p, e, batch_index, seg)
        embs_id.append(e)
    e = emb
    embs_reg = []
    for p in params['conv_reg']:
        e = _gravnet_layer(p, e, batch_index, seg)
        embs_reg.append(e)
    embedding_id = jnp.concatenate([x] + embs_id, axis=-1)
    preds_id = _ffn(embedding_id, params['nn_id'], 6)
    embedding_reg = jnp.concatenate([x] + embs_reg + [preds_id], axis=-1)
    preds_pt = _ffn(embedding_reg, params['nn_pt'], 1) + x[:, 1:2]
    preds_eta = _ffn(embedding_reg, params['nn_eta'], 1) + x[:, 2:3]
    preds_phi = _ffn(embedding_reg, params['nn_phi'], 1) + x[:, 3:4]
    preds_energy = _ffn(embedding_reg, params['nn_energy'], 1) + x[:, 4:5]
    preds_momentum = jnp.concatenate(
        [preds_pt, preds_eta, preds_phi, preds_energy], axis=-1)
    pred_charge = _ffn(embedding_reg, params['nn_charge'], 3)
    return (preds_id, preds_momentum, pred_charge)


# speedup vs baseline: 2.5895x; 1.0883x over previous
import functools

import jax
import jax.numpy as jnp
from jax import lax
from jax.experimental import pallas as pl
from jax.experimental.pallas import tpu as pltpu
from jax.experimental.pallas import tpu_sc as plsc

N = 10000
B = 4
K = 32
SPACE_DIMS = 4
ROW_TILE = 1000
GRID = N // ROW_TILE


def _elu(x):
    return jnp.where(x > 0, x, jnp.exp(x) - 1.0)


def _ln(x, g, b):
    mu = jnp.mean(x, axis=-1, keepdims=True)
    xc = x - mu
    var = jnp.mean(xc * xc, axis=-1, keepdims=True)
    return xc / jnp.sqrt(var + 1e-5) * g + b


def _dot(x, w):
    return jnp.dot(x, w, preferred_element_type=jnp.float32)


def _full_spec(shape):
    nd = len(shape)
    return pl.BlockSpec(shape, lambda i, _nd=nd: (0,) * _nd)


def _row_spec(d):
    return pl.BlockSpec((ROW_TILE, d), lambda i: (i, 0))


def _row_call(body, ins, consts, out_dims):
    in_specs = [_row_spec(a.shape[-1]) for a in ins]
    in_specs += [_full_spec(c.shape) for c in consts]
    if isinstance(out_dims, tuple):
        out_shape = tuple(jax.ShapeDtypeStruct((N, d), jnp.float32) for d in out_dims)
        out_specs = tuple(_row_spec(d) for d in out_dims)
    else:
        out_shape = jax.ShapeDtypeStruct((N, out_dims), jnp.float32)
        out_specs = _row_spec(out_dims)
    return pl.pallas_call(
        body,
        grid=(GRID,),
        in_specs=in_specs,
        out_specs=out_specs,
        out_shape=out_shape,
    )(*ins, *consts)


def _nn0_body(x_ref, w0, b0, w1, b1, w2, b2, w3, b3, o_ref):
    h = x_ref[...]
    h = _elu(_dot(h, w0[...]) + b0[...])
    h = _elu(_dot(h, w1[...]) + b1[...])
    h = _elu(_dot(h, w2[...]) + b2[...])
    o_ref[...] = _dot(h, w3[...]) + b3[...]


def _nn0(x, lins):
    consts = []
    for p in lins:
        consts += [p['w'], p['b'].reshape(1, -1)]
    return _row_call(_nn0_body, [x], consts, 128)


def _proj_body(e_ref, ws, bs, wh, bh, s_ref, h_ref, sq_ref):
    e = e_ref[...]
    s = _dot(e, ws[...]) + bs[...]
    s_ref[...] = s
    h = _dot(e, wh[...]) + bh[...]
    h_ref[...] = jnp.concatenate(
        [h, jnp.zeros((h.shape[0], 128 - h.shape[1]), jnp.float32)], axis=-1)
    sq_ref[...] = jnp.sum(s * s, axis=-1, keepdims=True)


def _proj(e, p):
    consts = [p['lin_s']['w'], p['lin_s']['b'].reshape(1, -1),
              p['lin_h']['w'], p['lin_h']['b'].reshape(1, -1)]
    return _row_call(_proj_body, [e], consts, (4, 128, 1))


def _gravout_body(e_ref, agg_ref, w, b, g, bb, o_ref):
    e = e_ref[...]
    xin = jnp.concatenate([e, agg_ref[...]], axis=-1)
    xn = _dot(xin, w[...]) + b[...]
    o_ref[...] = _ln(e + xn, g[...], bb[...])


def _gravout(e, agg, p):
    consts = [p['lin_out']['w'], p['lin_out']['b'].reshape(1, -1),
              p['ln']['g'].reshape(1, -1), p['ln']['b'].reshape(1, -1)]
    return _row_call(_gravout_body, [e, agg], consts, 128)


def _ffn_body(x_ref, *refs):
    o_ref = refs[-1]
    refs = refs[:-1]
    ws = refs[0:10]
    lns = refs[10:18]
    h = x_ref[...]
    for i in range(4):
        h = _elu(_dot(h, ws[2 * i][...]) + ws[2 * i + 1][...])
        h = _ln(h, lns[2 * i][...], lns[2 * i + 1][...])
    o_ref[...] = _dot(h, ws[8][...]) + ws[9][...]


def _ffn(x, p, dout):
    consts = []
    for lp in p['lins']:
        consts += [lp['w'], lp['b'].reshape(1, -1)]
    for lp in p['lns']:
        consts += [lp['g'].reshape(1, -1), lp['b'].reshape(1, -1)]
    return _row_call(_ffn_body, [x], consts, dout)


_UNROLL = 8
_BUF = 128 + 16 * _UNROLL
_RESEL_AT = 128
_NPAD = N + 16 * _UNROLL
_GRP = 16
_NQ = _GRP * K // 128


def _merge16(ka, va, kb, vb):
    kbr = lax.rev(kb, (0,))
    vbr = lax.rev(vb, (0,))
    m = ka <= kbr
    kl = jnp.where(m, ka, kbr)
    vl = jnp.where(m, va, vbr)
    kh = jnp.where(m, kbr, ka)
    vh = jnp.where(m, vbr, va)
    kl, vl = plsc.sort_key_val(kl, vl)
    kh, vh = plsc.sort_key_val(kh, vh)
    return kl, vl, kh, vh


def _sc_knn_agg(s_t, sq, h, seg):
    mesh = plsc.VectorSubcoreMesh(core_axis_name="c", subcore_axis_name="s")
    INF = float(jnp.inf)

    @functools.partial(
        pl.kernel,
        out_type=jax.ShapeDtypeStruct((N * 64,), jnp.float32),
        mesh=mesh,
        compiler_params=pltpu.CompilerParams(needs_layout_passes=False),
        scratch_types=[
            pltpu.VMEM((_NPAD,), jnp.float32),
            pltpu.VMEM((_NPAD,), jnp.float32),
            pltpu.VMEM((_NPAD,), jnp.float32),
            pltpu.VMEM((_NPAD,), jnp.float32),
            pltpu.VMEM((_NPAD,), jnp.float32),
            pltpu.VMEM((16,), jnp.float32),
            pltpu.VMEM((_BUF,), jnp.float32),
            pltpu.VMEM((_BUF,), jnp.int32),
            pltpu.VMEM((_GRP * K,), jnp.float32),
            pltpu.VMEM((_NQ, 128), jnp.int32),
            pltpu.VMEM((_NQ, 128, 128), jnp.float32),
            pltpu.VMEM((_GRP * 64,), jnp.float32),
            pltpu.SemaphoreType.DMA,
        ],
    )
    def body(s_t_hbm, sq_hbm, h_hbm, seg_hbm, out_hbm,
             s0, s1, s2, s3, sqv, segv, d2b, idxb, wbuf, gidx, hbuf, aggb,
             sem):
        iota = lax.iota(jnp.int32, 16)
        wid = lax.axis_index("s") * 2 + lax.axis_index("c")
        pltpu.sync_copy(s_t_hbm.at[0], s0)
        pltpu.sync_copy(s_t_hbm.at[1], s1)
        pltpu.sync_copy(s_t_hbm.at[2], s2)
        pltpu.sync_copy(s_t_hbm.at[3], s3)
        pltpu.sync_copy(sq_hbm, sqv)
        pltpu.sync_copy(seg_hbm, segv)
        zz = jnp.zeros((16,), jnp.int32)
        for q in range(_NQ):
            for o in range(8):
                gidx[q, pl.ds(o * 16, 16)] = zz

        ev = wid // 8
        sl = wid % 8
        sv = segv[pl.ds(0, 16)]
        st = jnp.sum(jnp.where(iota == ev, sv, 0.0)).astype(jnp.int32)
        en = jnp.sum(jnp.where(iota == ev + 1, sv, 0.0)).astype(jnp.int32)
        cnt_ev = en - st
        chunk = (cnt_ev + 7) // 8
        my_st = st + sl * chunk
        my_en = jnp.minimum(my_st + chunk, en)
        n_my = jnp.maximum(my_en - my_st, 0)
        ngrp = (n_my + _GRP - 1) // _GRP
        st16 = (st // 16) * 16
        nvec4 = (en - st16 + 16 * _UNROLL - 1) // (16 * _UNROLL)

        def reselect(cnt):
            ak = jnp.full((16,), INF, jnp.float32)
            av = jnp.zeros((16,), jnp.int32)
            bk = jnp.full((16,), INF, jnp.float32)
            bv = jnp.zeros((16,), jnp.int32)
            for b in range(_BUF // 16):
                off = b * 16
                kb = jnp.where(iota + off < cnt, d2b[pl.ds(off, 16)], INF)
                vb = idxb[pl.ds(off, 16)]
                kb, vb = plsc.sort_key_val(kb, vb)
                if b == 0:
                    ak, av = kb, vb
                else:
                    ak, av, hk, hv = _merge16(ak, av, kb, vb)
                    bk, bv, _, _ = _merge16(bk, bv, hk, hv)
            d2b[pl.ds(0, 16)] = ak
            idxb[pl.ds(0, 16)] = av
            d2b[pl.ds(16, 16)] = bk
            idxb[pl.ds(16, 16)] = bv
            return jnp.int32(K), jnp.max(bk)

        def select_row(i, r):
            iv = jnp.full((16,), 0, jnp.int32) + i
            s0i = plsc.load_gather(s0, [iv])
            s1i = plsc.load_gather(s1, [iv])
            s2i = plsc.load_gather(s2, [iv])
            s3i = plsc.load_gather(s3, [iv])
            sqi = plsc.load_gather(sqv, [iv])

            def cbody(v, carry):
                cnt, tau = carry
                jbase = st16 + v * (16 * _UNROLL)
                d2s, jvs, ms = [], [], []
                for u in range(_UNROLL):
                    j = jbase + u * 16
                    jv = j + iota
                    a0 = s0[pl.ds(j, 16)]
                    a1 = s1[pl.ds(j, 16)]
                    a2 = s2[pl.ds(j, 16)]
                    a3 = s3[pl.ds(j, 16)]
                    sqj = sqv[pl.ds(j, 16)]
                    t = a0 * s0i + a1 * s1i + a2 * s2i + a3 * s3i
                    d2 = (sqi + sqj) - 2.0 * t
                    valid = (jv >= st) & (jv < en)
                    d2 = jnp.where(valid, d2, INF)
                    d2s.append(d2)
                    jvs.append(jv)
                    ms.append(d2 < tau)
                pcs = [plsc.all_reduce_population_count(m)[0] for m in ms]
                off = cnt
                for u in range(_UNROLL):
                    plsc.store_compressed(d2b.at[pl.ds(off, 16)],
                                          d2s[u], mask=ms[u])
                    plsc.store_compressed(idxb.at[pl.ds(off, 16)],
                                          jvs[u], mask=ms[u])
                    off = off + pcs[u]
                cnt = off
                cnt, tau = lax.cond(cnt > _RESEL_AT,
                                    lambda c=cnt: reselect(c),
                                    lambda c=cnt, t=tau: (c, t))
                return cnt, tau

            cnt, _ = lax.fori_loop(0, nvec4, cbody, (jnp.int32(0), INF))
            reselect(cnt)
            q = r // 4
            for half in range(2):
                vv = idxb[pl.ds(half * 16, 16)]
                g0 = plsc.load_gather(s0, [vv])
                g1 = plsc.load_gather(s1, [vv])
                g2 = plsc.load_gather(s2, [vv])
                g3 = plsc.load_gather(s3, [vv])
                dd = ((s0i - g0) * (s0i - g0) + (s1i - g1) * (s1i - g1)
                      + (s2i - g2) * (s2i - g2) + (s3i - g3) * (s3i - g3))
                w = jnp.exp(-10.0 * dd)
                wbuf[pl.ds(r * K + half * 16, 16)] = w
                gidx[q, pl.ds((r % 4) * K + half * 16, 16)] = vv

        def agg_row(r):
            zf = jnp.zeros((16,), jnp.float32)
            ninf = jnp.full((16,), -INF, jnp.float32)

            def kbody(k4, carry):
                acc0, acc1, mx0, mx1 = carry
                hs, ws = [], []
                for kk in range(4):
                    slot = r * K + k4 * 4 + kk
                    q = slot // 128
                    rr = slot % 128
                    h0 = hbuf[q, rr, pl.ds(0, 16)]
                    h1 = hbuf[q, rr, pl.ds(16, 16)]
                    wk = plsc.load_gather(
                        wbuf, [jnp.full((16,), 0, jnp.int32) + slot])
                    hs.append((h0, h1))
                    ws.append(wk)
                for kk in range(4):
                    m0 = hs[kk][0] * ws[kk]
                    m1 = hs[kk][1] * ws[kk]
                    acc0 = acc0 + m0
                    acc1 = acc1 + m1
                    mx0 = jnp.maximum(mx0, m0)
                    mx1 = jnp.maximum(mx1, m1)
                return (acc0, acc1, mx0, mx1)

            acc0, acc1, mx0, mx1 = lax.fori_loop(
                0, K // 4, kbody, (zf, zf, ninf, ninf))
            sc = jnp.float32(1.0 / K)
            aggb[pl.ds(r * 64, 16)] = acc0 * sc
            aggb[pl.ds(r * 64 + 16, 16)] = acc1 * sc
            aggb[pl.ds(r * 64 + 32, 16)] = mx0
            aggb[pl.ds(r * 64 + 48, 16)] = mx1

        def gbody(g, _):
            base = my_st + g * _GRP
            nr = jnp.minimum(my_en - base, _GRP)

            copies = []
            for q in range(_NQ):
                def rbody(r, _, q=q):
                    rr = q * 4 + r

                    @pl.when(rr < nr)
                    def _():
                        select_row(base + rr, rr)
                    return 0

                lax.fori_loop(0, 4, rbody, 0)
                copies.append(
                    pltpu.async_copy(h_hbm.at[gidx.at[q]], hbuf.at[q], sem))
            for c in copies:
                c.wait()

            def abody(r, _):
                agg_row(r)
                return 0

            lax.fori_loop(0, nr, abody, 0)

            @pl.when(nr == _GRP)
            def _():
                pltpu.sync_copy(aggb, out_hbm.at[pl.ds(base * 64, _GRP * 64)])

            @pl.when(nr < _GRP)
            def _():
                def cb(r, _):
                    pltpu.sync_copy(aggb.at[pl.ds(r * 64, 64)],
                                    out_hbm.at[pl.ds((base + r) * 64, 64)])
                    return 0
                lax.fori_loop(0, nr, cb, 0)

            return 0

        lax.fori_loop(0, ngrp, gbody, 0)

    return body(s_t, sq, h, seg)


def _gravnet_layer(p, e, batch_index, seg):
    s, hh, sq = _proj(e, p)
    s_t = jnp.concatenate(
        [s.T, jnp.zeros((SPACE_DIMS, _NPAD - N), jnp.float32)], axis=1)
    sq_p = jnp.concatenate(
        [sq[:, 0], jnp.zeros((_NPAD - N,), jnp.float32)])
    agg = _sc_knn_agg(s_t, sq_p, hh, seg).reshape(N, 64)
    return _gravout(e, agg, p)


def kernel(x, params, batch_index):
    starts = jnp.searchsorted(batch_index, jnp.arange(B, dtype=jnp.int32),
                              side='left').astype(jnp.float32)
    seg = jnp.concatenate(
        [starts, jnp.full((12,), N, dtype=jnp.float32)])
    emb = _nn0(x, params['nn0'])
    e = emb
    embs_id = []
    for p in params['conv_id']:
        e = _gravnet_layer(p, e, batch_index, seg)
        embs_id.append(e)
    e = emb
    embs_reg = []
    for p in params['conv_reg']:
        e = _gravnet_layer(p, e, batch_index, seg)
        embs_reg.append(e)
    embedding_id = jnp.concatenate([x] + embs_id, axis=-1)
    preds_id = _ffn(embedding_id, params['nn_id'], 6)
    embedding_reg = jnp.concatenate([x] + embs_reg + [preds_id], axis=-1)
    preds_pt = _ffn(embedding_reg, params['nn_pt'], 1) + x[:, 1:2]
    preds_eta = _ffn(embedding_reg, params['nn_eta'], 1) + x[:, 2:3]
    preds_phi = _ffn(embedding_reg, params['nn_phi'], 1) + x[:, 3:4]
    preds_energy = _ffn(embedding_reg, params['nn_energy'], 1) + x[:, 4:5]
    preds_momentum = jnp.concatenate(
        [preds_pt, preds_eta, preds_phi, preds_energy], axis=-1)
    pred_charge = _ffn(embedding_reg, params['nn_charge'], 3)
    return (preds_id, preds_momentum, pred_charge)


# speedup vs baseline: 3.3699x; 1.3014x over previous
import functools

import jax
import jax.numpy as jnp
from jax import lax
from jax.experimental import pallas as pl
from jax.experimental.pallas import tpu as pltpu
from jax.experimental.pallas import tpu_sc as plsc

N = 10000
B = 4
K = 32
SPACE_DIMS = 4
ROW_TILE = 1000
GRID = N // ROW_TILE


def _elu(x):
    return jnp.where(x > 0, x, jnp.exp(x) - 1.0)


def _ln(x, g, b):
    mu = jnp.mean(x, axis=-1, keepdims=True)
    xc = x - mu
    var = jnp.mean(xc * xc, axis=-1, keepdims=True)
    return xc / jnp.sqrt(var + 1e-5) * g + b


def _dot(x, w):
    return jnp.dot(x, w, preferred_element_type=jnp.float32)


def _full_spec(shape):
    nd = len(shape)
    return pl.BlockSpec(shape, lambda i, _nd=nd: (0,) * _nd)


def _row_spec(d):
    return pl.BlockSpec((ROW_TILE, d), lambda i: (i, 0))


def _row_call(body, ins, consts, out_dims):
    in_specs = [_row_spec(a.shape[-1]) for a in ins]
    in_specs += [_full_spec(c.shape) for c in consts]
    if isinstance(out_dims, tuple):
        out_shape = tuple(jax.ShapeDtypeStruct((N, d), jnp.float32) for d in out_dims)
        out_specs = tuple(_row_spec(d) for d in out_dims)
    else:
        out_shape = jax.ShapeDtypeStruct((N, out_dims), jnp.float32)
        out_specs = _row_spec(out_dims)
    return pl.pallas_call(
        body,
        grid=(GRID,),
        in_specs=in_specs,
        out_specs=out_specs,
        out_shape=out_shape,
    )(*ins, *consts)


def _nn0_body(x_ref, w0, b0, w1, b1, w2, b2, w3, b3, o_ref):
    h = x_ref[...]
    h = _elu(_dot(h, w0[...]) + b0[...])
    h = _elu(_dot(h, w1[...]) + b1[...])
    h = _elu(_dot(h, w2[...]) + b2[...])
    o_ref[...] = _dot(h, w3[...]) + b3[...]


def _nn0(x, lins):
    consts = []
    for p in lins:
        consts += [p['w'], p['b'].reshape(1, -1)]
    return _row_call(_nn0_body, [x], consts, 128)


def _proj_body(e_ref, ws, bs, wh, bh, s_ref, h_ref, sq_ref):
    e = e_ref[...]
    s = _dot(e, ws[...]) + bs[...]
    s_ref[...] = s
    h = _dot(e, wh[...]) + bh[...]
    h_ref[...] = jnp.concatenate(
        [h, jnp.zeros((h.shape[0], 128 - h.shape[1]), jnp.float32)], axis=-1)
    sq_ref[...] = jnp.sum(s * s, axis=-1, keepdims=True)


def _proj(e, p):
    consts = [p['lin_s']['w'], p['lin_s']['b'].reshape(1, -1),
              p['lin_h']['w'], p['lin_h']['b'].reshape(1, -1)]
    return _row_call(_proj_body, [e], consts, (4, 128, 1))


def _gravout_body(e_ref, agg_ref, w, b, g, bb, o_ref):
    e = e_ref[...]
    xin = jnp.concatenate([e, agg_ref[...]], axis=-1)
    xn = _dot(xin, w[...]) + b[...]
    o_ref[...] = _ln(e + xn, g[...], bb[...])


def _gravout(e, agg, p):
    consts = [p['lin_out']['w'], p['lin_out']['b'].reshape(1, -1),
              p['ln']['g'].reshape(1, -1), p['ln']['b'].reshape(1, -1)]
    return _row_call(_gravout_body, [e, agg], consts, 128)


def _ffn_body(x_ref, *refs):
    o_ref = refs[-1]
    refs = refs[:-1]
    ws = refs[0:10]
    lns = refs[10:18]
    h = x_ref[...]
    for i in range(4):
        h = _elu(_dot(h, ws[2 * i][...]) + ws[2 * i + 1][...])
        h = _ln(h, lns[2 * i][...], lns[2 * i + 1][...])
    o_ref[...] = _dot(h, ws[8][...]) + ws[9][...]


def _ffn(x, p, dout):
    consts = []
    for lp in p['lins']:
        consts += [lp['w'], lp['b'].reshape(1, -1)]
    for lp in p['lns']:
        consts += [lp['g'].reshape(1, -1), lp['b'].reshape(1, -1)]
    return _row_call(_ffn_body, [x], consts, dout)


_UNROLL = 8
_BUF = 128 + 16 * _UNROLL
_RESEL_AT = 128
_NPAD = N + 16 * _UNROLL
_GRP = 16
_NQ = _GRP * K // 128


def _merge16(ka, va, kb, vb):
    kbr = lax.rev(kb, (0,))
    vbr = lax.rev(vb, (0,))
    m = ka <= kbr
    kl = jnp.where(m, ka, kbr)
    vl = jnp.where(m, va, vbr)
    kh = jnp.where(m, kbr, ka)
    vh = jnp.where(m, vbr, va)
    kl, vl = plsc.sort_key_val(kl, vl)
    kh, vh = plsc.sort_key_val(kh, vh)
    return kl, vl, kh, vh


def _sc_knn_agg(s_t, sq, h, seg):
    mesh = plsc.VectorSubcoreMesh(core_axis_name="c", subcore_axis_name="s")
    INF = float(jnp.inf)

    @functools.partial(
        pl.kernel,
        out_type=jax.ShapeDtypeStruct((N * 64,), jnp.float32),
        mesh=mesh,
        compiler_params=pltpu.CompilerParams(needs_layout_passes=False),
        scratch_types=[
            pltpu.VMEM((_NPAD,), jnp.float32),
            pltpu.VMEM((_NPAD,), jnp.float32),
            pltpu.VMEM((_NPAD,), jnp.float32),
            pltpu.VMEM((_NPAD,), jnp.float32),
            pltpu.VMEM((_NPAD,), jnp.float32),
            pltpu.VMEM((16,), jnp.float32),
            pltpu.VMEM((_BUF,), jnp.float32),
            pltpu.VMEM((_BUF,), jnp.int32),
            pltpu.VMEM((_GRP * K,), jnp.float32),
            pltpu.VMEM((_NQ, 128), jnp.int32),
            pltpu.VMEM((_NQ, 128, 128), jnp.float32),
            pltpu.VMEM((_GRP * 64,), jnp.float32),
            pltpu.SemaphoreType.DMA,
        ],
    )
    def body(s_t_hbm, sq_hbm, h_hbm, seg_hbm, out_hbm,
             s0, s1, s2, s3, sqv, segv, d2b, idxb, wbuf, gidx, hbuf, aggb,
             sem):
        iota = lax.iota(jnp.int32, 16)
        wid = lax.axis_index("s") * 2 + lax.axis_index("c")
        pltpu.sync_copy(s_t_hbm.at[0], s0)
        pltpu.sync_copy(s_t_hbm.at[1], s1)
        pltpu.sync_copy(s_t_hbm.at[2], s2)
        pltpu.sync_copy(s_t_hbm.at[3], s3)
        pltpu.sync_copy(sq_hbm, sqv)
        pltpu.sync_copy(seg_hbm, segv)
        zz = jnp.zeros((16,), jnp.int32)
        for q in range(_NQ):
            for o in range(8):
                gidx[q, pl.ds(o * 16, 16)] = zz

        ev = wid // 8
        sl = wid % 8
        sv = segv[pl.ds(0, 16)]
        st = jnp.sum(jnp.where(iota == ev, sv, 0.0)).astype(jnp.int32)
        en = jnp.sum(jnp.where(iota == ev + 1, sv, 0.0)).astype(jnp.int32)
        cnt_ev = en - st
        chunk = (cnt_ev + 7) // 8
        my_st = st + sl * chunk
        my_en = jnp.minimum(my_st + chunk, en)
        n_my = jnp.maximum(my_en - my_st, 0)
        ngrp = (n_my + _GRP - 1) // _GRP
        st16 = (st // 16) * 16
        nvec4 = (en - st16 + 16 * _UNROLL - 1) // (16 * _UNROLL)

        def reselect(cnt):
            ks, vs = [], []
            for b in range(_BUF // 16):
                off = b * 16
                kb = jnp.where(iota + off < cnt, d2b[pl.ds(off, 16)], INF)
                vb = idxb[pl.ds(off, 16)]
                kb, vb = plsc.sort_key_val(kb, vb)
                ks.append(kb)
                vs.append(vb)
            runs = []
            for i in range(0, len(ks), 2):
                runs.append(_merge16(ks[i], vs[i], ks[i + 1], vs[i + 1]))
            while len(runs) > 1:
                nxt = []
                for i in range(0, len(runs), 2):
                    a0, a0v, a1, a1v = runs[i]
                    b0, b0v, b1, b1v = runs[i + 1]
                    l0, l0v, h0, h0v = _merge16(a0, a0v, b0, b0v)
                    l1, l1v, _, _ = _merge16(a1, a1v, b1, b1v)
                    m, mv, _, _ = _merge16(h0, h0v, l1, l1v)
                    nxt.append((l0, l0v, m, mv))
                runs = nxt
            ak, av, bk, bv = runs[0]
            d2b[pl.ds(0, 16)] = ak
            idxb[pl.ds(0, 16)] = av
            d2b[pl.ds(16, 16)] = bk
            idxb[pl.ds(16, 16)] = bv
            return jnp.int32(K), jnp.max(bk)

        def select_row(i, r):
            iv = jnp.full((16,), 0, jnp.int32) + i
            s0i = plsc.load_gather(s0, [iv])
            s1i = plsc.load_gather(s1, [iv])
            s2i = plsc.load_gather(s2, [iv])
            s3i = plsc.load_gather(s3, [iv])
            sqi = plsc.load_gather(sqv, [iv])

            def cbody(v, carry):
                cnt, tau = carry
                jbase = st16 + v * (16 * _UNROLL)
                d2s, jvs, ms = [], [], []
                for u in range(_UNROLL):
                    j = jbase + u * 16
                    jv = j + iota
                    a0 = s0[pl.ds(j, 16)]
                    a1 = s1[pl.ds(j, 16)]
                    a2 = s2[pl.ds(j, 16)]
                    a3 = s3[pl.ds(j, 16)]
                    sqj = sqv[pl.ds(j, 16)]
                    t = a0 * s0i + a1 * s1i + a2 * s2i + a3 * s3i
                    d2 = (sqi + sqj) - 2.0 * t
                    valid = (jv >= st) & (jv < en)
                    d2 = jnp.where(valid, d2, INF)
                    d2s.append(d2)
                    jvs.append(jv)
                    ms.append(d2 < tau)
                pcs = [plsc.all_reduce_population_count(m)[0] for m in ms]
                off = cnt
                for u in range(_UNROLL):
                    plsc.store_compressed(d2b.at[pl.ds(off, 16)],
                                          d2s[u], mask=ms[u])
                    plsc.store_compressed(idxb.at[pl.ds(off, 16)],
                                          jvs[u], mask=ms[u])
                    off = off + pcs[u]
                cnt = off
                cnt, tau = lax.cond(cnt > _RESEL_AT,
                                    lambda c=cnt: reselect(c),
                                    lambda c=cnt, t=tau: (c, t))
                return cnt, tau

            cnt, _ = lax.fori_loop(0, nvec4, cbody, (jnp.int32(0), INF))
            reselect(cnt)
            q = r // 4
            for half in range(2):
                vv = idxb[pl.ds(half * 16, 16)]
                g0 = plsc.load_gather(s0, [vv])
                g1 = plsc.load_gather(s1, [vv])
                g2 = plsc.load_gather(s2, [vv])
                g3 = plsc.load_gather(s3, [vv])
                dd = ((s0i - g0) * (s0i - g0) + (s1i - g1) * (s1i - g1)
                      + (s2i - g2) * (s2i - g2) + (s3i - g3) * (s3i - g3))
                w = jnp.exp(-10.0 * dd)
                wbuf[pl.ds(r * K + half * 16, 16)] = w
                gidx[q, pl.ds((r % 4) * K + half * 16, 16)] = vv

        def agg_row(r):
            zf = jnp.zeros((16,), jnp.float32)
            ninf = jnp.full((16,), -INF, jnp.float32)

            def kbody(k4, carry):
                acc0, acc1, mx0, mx1 = carry
                hs, ws = [], []
                for kk in range(4):
                    slot = r * K + k4 * 4 + kk
                    q = slot // 128
                    rr = slot % 128
                    h0 = hbuf[q, rr, pl.ds(0, 16)]
                    h1 = hbuf[q, rr, pl.ds(16, 16)]
                    wk = plsc.load_gather(
                        wbuf, [jnp.full((16,), 0, jnp.int32) + slot])
                    hs.append((h0, h1))
                    ws.append(wk)
                for kk in range(4):
                    m0 = hs[kk][0] * ws[kk]
                    m1 = hs[kk][1] * ws[kk]
                    acc0 = acc0 + m0
                    acc1 = acc1 + m1
                    mx0 = jnp.maximum(mx0, m0)
                    mx1 = jnp.maximum(mx1, m1)
                return (acc0, acc1, mx0, mx1)

            acc0, acc1, mx0, mx1 = lax.fori_loop(
                0, K // 4, kbody, (zf, zf, ninf, ninf))
            sc = jnp.float32(1.0 / K)
            aggb[pl.ds(r * 64, 16)] = acc0 * sc
            aggb[pl.ds(r * 64 + 16, 16)] = acc1 * sc
            aggb[pl.ds(r * 64 + 32, 16)] = mx0
            aggb[pl.ds(r * 64 + 48, 16)] = mx1

        def gbody(g, _):
            base = my_st + g * _GRP
            nr = jnp.minimum(my_en - base, _GRP)

            copies = []
            for q in range(_NQ):
                def rbody(r, _, q=q):
                    rr = q * 4 + r

                    @pl.when(rr < nr)
                    def _():
                        select_row(base + rr, rr)
                    return 0

                lax.fori_loop(0, 4, rbody, 0)
                copies.append(
                    pltpu.async_copy(h_hbm.at[gidx.at[q]], hbuf.at[q], sem))
            for c in copies:
                c.wait()

            def abody(r, _):
                agg_row(r)
                return 0

            lax.fori_loop(0, nr, abody, 0)

            @pl.when(nr == _GRP)
            def _():
                pltpu.sync_copy(aggb, out_hbm.at[pl.ds(base * 64, _GRP * 64)])

            @pl.when(nr < _GRP)
            def _():
                def cb(r, _):
                    pltpu.sync_copy(aggb.at[pl.ds(r * 64, 64)],
                                    out_hbm.at[pl.ds((base + r) * 64, 64)])
                    return 0
                lax.fori_loop(0, nr, cb, 0)

            return 0

        lax.fori_loop(0, ngrp, gbody, 0)

    return body(s_t, sq, h, seg)


def _gravnet_layer(p, e, batch_index, seg):
    s, hh, sq = _proj(e, p)
    s_t = jnp.concatenate(
        [s.T, jnp.zeros((SPACE_DIMS, _NPAD - N), jnp.float32)], axis=1)
    sq_p = jnp.concatenate(
        [sq[:, 0], jnp.zeros((_NPAD - N,), jnp.float32)])
    agg = _sc_knn_agg(s_t, sq_p, hh, seg).reshape(N, 64)
    return _gravout(e, agg, p)


def kernel(x, params, batch_index):
    starts = jnp.searchsorted(batch_index, jnp.arange(B, dtype=jnp.int32),
                              side='left').astype(jnp.float32)
    seg = jnp.concatenate(
        [starts, jnp.full((12,), N, dtype=jnp.float32)])
    emb = _nn0(x, params['nn0'])
    e = emb
    embs_id = []
    for p in params['conv_id']:
        e = _gravnet_layer(p, e, batch_index, seg)
        embs_id.append(e)
    e = emb
    embs_reg = []
    for p in params['conv_reg']:
        e = _gravnet_layer(p, e, batch_index, seg)
        embs_reg.append(e)
    embedding_id = jnp.concatenate([x] + embs_id, axis=-1)
    preds_id = _ffn(embedding_id, params['nn_id'], 6)
    embedding_reg = jnp.concatenate([x] + embs_reg + [preds_id], axis=-1)
    preds_pt = _ffn(embedding_reg, params['nn_pt'], 1) + x[:, 1:2]
    preds_eta = _ffn(embedding_reg, params['nn_eta'], 1) + x[:, 2:3]
    preds_phi = _ffn(embedding_reg, params['nn_phi'], 1) + x[:, 3:4]
    preds_energy = _ffn(embedding_reg, params['nn_energy'], 1) + x[:, 4:5]
    preds_momentum = jnp.concatenate(
        [preds_pt, preds_eta, preds_phi, preds_energy], axis=-1)
    pred_charge = _ffn(embedding_reg, params['nn_charge'], 3)
    return (preds_id, preds_momentum, pred_charge)


# speedup vs baseline: 3.6798x; 1.0920x over previous
import functools

import jax
import jax.numpy as jnp
from jax import lax
from jax.experimental import pallas as pl
from jax.experimental.pallas import tpu as pltpu
from jax.experimental.pallas import tpu_sc as plsc

N = 10000
B = 4
K = 32
SPACE_DIMS = 4
ROW_TILE = 1000
GRID = N // ROW_TILE


def _elu(x):
    return jnp.where(x > 0, x, jnp.exp(x) - 1.0)


def _ln(x, g, b):
    mu = jnp.mean(x, axis=-1, keepdims=True)
    xc = x - mu
    var = jnp.mean(xc * xc, axis=-1, keepdims=True)
    return xc / jnp.sqrt(var + 1e-5) * g + b


def _dot(x, w):
    return jnp.dot(x, w, preferred_element_type=jnp.float32)


def _full_spec(shape):
    nd = len(shape)
    return pl.BlockSpec(shape, lambda i, _nd=nd: (0,) * _nd)


def _row_spec(d):
    return pl.BlockSpec((ROW_TILE, d), lambda i: (i, 0))


def _row_call(body, ins, consts, out_dims):
    in_specs = [_row_spec(a.shape[-1]) for a in ins]
    in_specs += [_full_spec(c.shape) for c in consts]
    if isinstance(out_dims, tuple):
        out_shape = tuple(jax.ShapeDtypeStruct((N, d), jnp.float32) for d in out_dims)
        out_specs = tuple(_row_spec(d) for d in out_dims)
    else:
        out_shape = jax.ShapeDtypeStruct((N, out_dims), jnp.float32)
        out_specs = _row_spec(out_dims)
    return pl.pallas_call(
        body,
        grid=(GRID,),
        in_specs=in_specs,
        out_specs=out_specs,
        out_shape=out_shape,
    )(*ins, *consts)


def _nn0_body(x_ref, w0, b0, w1, b1, w2, b2, w3, b3, o_ref):
    h = x_ref[...]
    h = _elu(_dot(h, w0[...]) + b0[...])
    h = _elu(_dot(h, w1[...]) + b1[...])
    h = _elu(_dot(h, w2[...]) + b2[...])
    o_ref[...] = _dot(h, w3[...]) + b3[...]


def _nn0(x, lins):
    consts = []
    for p in lins:
        consts += [p['w'], p['b'].reshape(1, -1)]
    return _row_call(_nn0_body, [x], consts, 128)


def _proj_body(e_ref, ws, bs, wh, bh, s_ref, h_ref, sq_ref):
    e = e_ref[...]
    s = _dot(e, ws[...]) + bs[...]
    s_ref[...] = s
    h = _dot(e, wh[...]) + bh[...]
    h_ref[...] = jnp.concatenate(
        [h, jnp.zeros((h.shape[0], 128 - h.shape[1]), jnp.float32)], axis=-1)
    sq_ref[...] = jnp.sum(s * s, axis=-1, keepdims=True)


def _proj(e, p):
    consts = [p['lin_s']['w'], p['lin_s']['b'].reshape(1, -1),
              p['lin_h']['w'], p['lin_h']['b'].reshape(1, -1)]
    return _row_call(_proj_body, [e], consts, (4, 128, 1))


def _gravout_body(e_ref, agg_ref, w, b, g, bb, o_ref):
    e = e_ref[...]
    xin = jnp.concatenate([e, agg_ref[...]], axis=-1)
    xn = _dot(xin, w[...]) + b[...]
    o_ref[...] = _ln(e + xn, g[...], bb[...])


def _gravout(e, agg, p):
    consts = [p['lin_out']['w'], p['lin_out']['b'].reshape(1, -1),
              p['ln']['g'].reshape(1, -1), p['ln']['b'].reshape(1, -1)]
    return _row_call(_gravout_body, [e, agg], consts, 128)


def _ffn_body(x_ref, *refs):
    o_ref = refs[-1]
    refs = refs[:-1]
    ws = refs[0:10]
    lns = refs[10:18]
    h = x_ref[...]
    for i in range(4):
        h = _elu(_dot(h, ws[2 * i][...]) + ws[2 * i + 1][...])
        h = _ln(h, lns[2 * i][...], lns[2 * i + 1][...])
    o_ref[...] = _dot(h, ws[8][...]) + ws[9][...]


def _ffn(x, p, dout):
    consts = []
    for lp in p['lins']:
        consts += [lp['w'], lp['b'].reshape(1, -1)]
    for lp in p['lns']:
        consts += [lp['g'].reshape(1, -1), lp['b'].reshape(1, -1)]
    return _row_call(_ffn_body, [x], consts, dout)


_UNROLL = 8
_BUF = 128 + 16 * _UNROLL
_RESEL_AT = 128
_NPAD = N + 16 * (_UNROLL + 2)
_GRP = 16
_NQ = _GRP * K // 128


def _merge16(ka, va, kb, vb):
    kbr = lax.rev(kb, (0,))
    vbr = lax.rev(vb, (0,))
    m = ka <= kbr
    kl = jnp.where(m, ka, kbr)
    vl = jnp.where(m, va, vbr)
    kh = jnp.where(m, kbr, ka)
    vh = jnp.where(m, vbr, va)
    kl, vl = plsc.sort_key_val(kl, vl)
    kh, vh = plsc.sort_key_val(kh, vh)
    return kl, vl, kh, vh


def _sc_knn_agg(s_t, sq, h, seg):
    mesh = plsc.VectorSubcoreMesh(core_axis_name="c", subcore_axis_name="s")
    INF = float(jnp.inf)

    @functools.partial(
        pl.kernel,
        out_type=jax.ShapeDtypeStruct((N * 64,), jnp.float32),
        mesh=mesh,
        compiler_params=pltpu.CompilerParams(needs_layout_passes=False),
        scratch_types=[
            pltpu.VMEM((_NPAD,), jnp.float32),
            pltpu.VMEM((_NPAD,), jnp.float32),
            pltpu.VMEM((_NPAD,), jnp.float32),
            pltpu.VMEM((_NPAD,), jnp.float32),
            pltpu.VMEM((_NPAD,), jnp.float32),
            pltpu.VMEM((16,), jnp.float32),
            pltpu.VMEM((_BUF,), jnp.float32),
            pltpu.VMEM((_BUF,), jnp.int32),
            pltpu.VMEM((_GRP * K,), jnp.float32),
            pltpu.VMEM((_NQ, 128), jnp.int32),
            pltpu.VMEM((_NQ, 128, 128), jnp.float32),
            pltpu.VMEM((_GRP * 64,), jnp.float32),
            pltpu.SemaphoreType.DMA,
        ],
    )
    def body(s_t_hbm, sq_hbm, h_hbm, seg_hbm, out_hbm,
             s0, s1, s2, s3, sqv, segv, d2b, idxb, wbuf, gidx, hbuf, aggb,
             sem):
        iota = lax.iota(jnp.int32, 16)
        wid = lax.axis_index("s") * 2 + lax.axis_index("c")
        pltpu.sync_copy(s_t_hbm.at[0], s0)
        pltpu.sync_copy(s_t_hbm.at[1], s1)
        pltpu.sync_copy(s_t_hbm.at[2], s2)
        pltpu.sync_copy(s_t_hbm.at[3], s3)
        pltpu.sync_copy(sq_hbm, sqv)
        pltpu.sync_copy(seg_hbm, segv)
        zz = jnp.zeros((16,), jnp.int32)
        for q in range(_NQ):
            for o in range(8):
                gidx[q, pl.ds(o * 16, 16)] = zz

        ev = wid // 8
        sl = wid % 8
        sv = segv[pl.ds(0, 16)]
        st = jnp.sum(jnp.where(iota == ev, sv, 0.0)).astype(jnp.int32)
        en = jnp.sum(jnp.where(iota == ev + 1, sv, 0.0)).astype(jnp.int32)
        cnt_ev = en - st
        chunk = (cnt_ev + 7) // 8
        my_st = st + sl * chunk
        my_en = jnp.minimum(my_st + chunk, en)
        n_my = jnp.maximum(my_en - my_st, 0)
        ngrp = (n_my + _GRP - 1) // _GRP
        st16 = (st // 16) * 16
        nvec4 = (en - st16 + 16 * _UNROLL - 1) // (16 * _UNROLL)

        hv = sqv[pl.ds(st16, 16)]
        sqv[pl.ds(st16, 16)] = jnp.where(st16 + iota < st, INF, hv)
        e0 = (en // 16) * 16
        for b in range(_UNROLL + 2):
            j = e0 + b * 16
            tv = sqv[pl.ds(j, 16)]
            sqv[pl.ds(j, 16)] = jnp.where(j + iota >= en, INF, tv)

        def reselect(cnt):
            ks, vs = [], []
            for b in range(_BUF // 16):
                off = b * 16
                kb = jnp.where(iota + off < cnt, d2b[pl.ds(off, 16)], INF)
                vb = idxb[pl.ds(off, 16)]
                kb, vb = plsc.sort_key_val(kb, vb)
                ks.append(kb)
                vs.append(vb)
            runs = []
            for i in range(0, len(ks), 2):
                runs.append(_merge16(ks[i], vs[i], ks[i + 1], vs[i + 1]))
            while len(runs) > 1:
                nxt = []
                for i in range(0, len(runs), 2):
                    a0, a0v, a1, a1v = runs[i]
                    b0, b0v, b1, b1v = runs[i + 1]
                    l0, l0v, h0, h0v = _merge16(a0, a0v, b0, b0v)
                    l1, l1v, _, _ = _merge16(a1, a1v, b1, b1v)
                    m, mv, _, _ = _merge16(h0, h0v, l1, l1v)
                    nxt.append((l0, l0v, m, mv))
                runs = nxt
            ak, av, bk, bv = runs[0]
            d2b[pl.ds(0, 16)] = ak
            idxb[pl.ds(0, 16)] = av
            d2b[pl.ds(16, 16)] = bk
            idxb[pl.ds(16, 16)] = bv
            return jnp.int32(K), jnp.max(bk)

        def select_row(i, r):
            iv = jnp.full((16,), 0, jnp.int32) + i
            s0i = plsc.load_gather(s0, [iv])
            s1i = plsc.load_gather(s1, [iv])
            s2i = plsc.load_gather(s2, [iv])
            s3i = plsc.load_gather(s3, [iv])
            sqi = plsc.load_gather(sqv, [iv])
            n0 = -2.0 * s0i
            n1 = -2.0 * s1i
            n2 = -2.0 * s2i
            n3 = -2.0 * s3i

            def cbody(v, carry):
                cnt, tau = carry
                jbase = st16 + v * (16 * _UNROLL)
                d2s, jvs, ms = [], [], []
                for u in range(_UNROLL):
                    j = jbase + u * 16
                    jv = j + iota
                    a0 = s0[pl.ds(j, 16)]
                    a1 = s1[pl.ds(j, 16)]
                    a2 = s2[pl.ds(j, 16)]
                    a3 = s3[pl.ds(j, 16)]
                    sqj = sqv[pl.ds(j, 16)]
                    t = a0 * n0 + a1 * n1 + a2 * n2 + a3 * n3
                    d2 = (sqi + sqj) + t
                    d2s.append(d2)
                    jvs.append(jv)
                    ms.append(d2 < tau)
                pcs = [plsc.all_reduce_population_count(m)[0] for m in ms]
                off = cnt
                for u in range(_UNROLL):
                    plsc.store_compressed(d2b.at[pl.ds(off, 16)],
                                          d2s[u], mask=ms[u])
                    plsc.store_compressed(idxb.at[pl.ds(off, 16)],
                                          jvs[u], mask=ms[u])
                    off = off + pcs[u]
                cnt = off
                cnt, tau = lax.cond(cnt > _RESEL_AT,
                                    lambda c=cnt: reselect(c),
                                    lambda c=cnt, t=tau: (c, t))
                return cnt, tau

            cnt, _ = lax.fori_loop(0, nvec4, cbody, (jnp.int32(0), INF))
            reselect(cnt)
            q = r // 4
            for half in range(2):
                vv = idxb[pl.ds(half * 16, 16)]
                g0 = plsc.load_gather(s0, [vv])
                g1 = plsc.load_gather(s1, [vv])
                g2 = plsc.load_gather(s2, [vv])
                g3 = plsc.load_gather(s3, [vv])
                dd = ((s0i - g0) * (s0i - g0) + (s1i - g1) * (s1i - g1)
                      + (s2i - g2) * (s2i - g2) + (s3i - g3) * (s3i - g3))
                w = jnp.exp(-10.0 * dd)
                wbuf[pl.ds(r * K + half * 16, 16)] = w
                gidx[q, pl.ds((r % 4) * K + half * 16, 16)] = vv

        def agg_row(r):
            zf = jnp.zeros((16,), jnp.float32)
            ninf = jnp.full((16,), -INF, jnp.float32)

            def kbody(k4, carry):
                acc0, acc1, mx0, mx1 = carry
                hs, ws = [], []
                for kk in range(4):
                    slot = r * K + k4 * 4 + kk
                    q = slot // 128
                    rr = slot % 128
                    h0 = hbuf[q, rr, pl.ds(0, 16)]
                    h1 = hbuf[q, rr, pl.ds(16, 16)]
                    wk = plsc.load_gather(
                        wbuf, [jnp.full((16,), 0, jnp.int32) + slot])
                    hs.append((h0, h1))
                    ws.append(wk)
                for kk in range(4):
                    m0 = hs[kk][0] * ws[kk]
                    m1 = hs[kk][1] * ws[kk]
                    acc0 = acc0 + m0
                    acc1 = acc1 + m1
                    mx0 = jnp.maximum(mx0, m0)
                    mx1 = jnp.maximum(mx1, m1)
                return (acc0, acc1, mx0, mx1)

            acc0, acc1, mx0, mx1 = lax.fori_loop(
                0, K // 4, kbody, (zf, zf, ninf, ninf))
            sc = jnp.float32(1.0 / K)
            aggb[pl.ds(r * 64, 16)] = acc0 * sc
            aggb[pl.ds(r * 64 + 16, 16)] = acc1 * sc
            aggb[pl.ds(r * 64 + 32, 16)] = mx0
            aggb[pl.ds(r * 64 + 48, 16)] = mx1

        def gbody(g, _):
            base = my_st + g * _GRP
            nr = jnp.minimum(my_en - base, _GRP)

            copies = []
            for q in range(_NQ):
                def rbody(r, _, q=q):
                    rr = q * 4 + r

                    @pl.when(rr < nr)
                    def _():
                        select_row(base + rr, rr)
                    return 0

                lax.fori_loop(0, 4, rbody, 0)
                copies.append(
                    pltpu.async_copy(h_hbm.at[gidx.at[q]], hbuf.at[q], sem))
            for c in copies:
                c.wait()

            def abody(r, _):
                agg_row(r)
                return 0

            lax.fori_loop(0, nr, abody, 0)

            @pl.when(nr == _GRP)
            def _():
                pltpu.sync_copy(aggb, out_hbm.at[pl.ds(base * 64, _GRP * 64)])

            @pl.when(nr < _GRP)
            def _():
                def cb(r, _):
                    pltpu.sync_copy(aggb.at[pl.ds(r * 64, 64)],
                                    out_hbm.at[pl.ds((base + r) * 64, 64)])
                    return 0
                lax.fori_loop(0, nr, cb, 0)

            return 0

        lax.fori_loop(0, ngrp, gbody, 0)

    return body(s_t, sq, h, seg)


def _gravnet_layer(p, e, batch_index, seg):
    s, hh, sq = _proj(e, p)
    s_t = jnp.concatenate(
        [s.T, jnp.zeros((SPACE_DIMS, _NPAD - N), jnp.float32)], axis=1)
    sq_p = jnp.concatenate(
        [sq[:, 0], jnp.zeros((_NPAD - N,), jnp.float32)])
    agg = _sc_knn_agg(s_t, sq_p, hh, seg).reshape(N, 64)
    return _gravout(e, agg, p)


def kernel(x, params, batch_index):
    starts = jnp.searchsorted(batch_index, jnp.arange(B, dtype=jnp.int32),
                              side='left').astype(jnp.float32)
    seg = jnp.concatenate(
        [starts, jnp.full((12,), N, dtype=jnp.float32)])
    emb = _nn0(x, params['nn0'])
    e = emb
    embs_id = []
    for p in params['conv_id']:
        e = _gravnet_layer(p, e, batch_index, seg)
        embs_id.append(e)
    e = emb
    embs_reg = []
    for p in params['conv_reg']:
        e = _gravnet_layer(p, e, batch_index, seg)
        embs_reg.append(e)
    embedding_id = jnp.concatenate([x] + embs_id, axis=-1)
    preds_id = _ffn(embedding_id, params['nn_id'], 6)
    embedding_reg = jnp.concatenate([x] + embs_reg + [preds_id], axis=-1)
    preds_pt = _ffn(embedding_reg, params['nn_pt'], 1) + x[:, 1:2]
    preds_eta = _ffn(embedding_reg, params['nn_eta'], 1) + x[:, 2:3]
    preds_phi = _ffn(embedding_reg, params['nn_phi'], 1) + x[:, 3:4]
    preds_energy = _ffn(embedding_reg, params['nn_energy'], 1) + x[:, 4:5]
    preds_momentum = jnp.concatenate(
        [preds_pt, preds_eta, preds_phi, preds_energy], axis=-1)
    pred_charge = _ffn(embedding_reg, params['nn_charge'], 3)
    return (preds_id, preds_momentum, pred_charge)
